# Initial kernel scaffold; baseline (speedup 1.0000x reference)
import functools

import jax
import jax.numpy as jnp
from jax import lax
from jax.experimental import pallas as pl
from jax.experimental.pallas import tpu as pltpu
from jax.experimental.pallas import tpu_sc as plsc

_NC = 2
_NS = 16
_LANES = 16


def _cbow_sparsecore(indices, emb_table):
    B, L = indices.shape
    V, D = emb_table.shape
    NW = _NC * _NS
    BPW = B // NW
    CB = 2
    ROWS = CB * L
    NCHUNK = BPW // CB

    idx3 = indices.reshape(NW, NCHUNK, ROWS)
    mesh = plsc.VectorSubcoreMesh(
        core_axis_name="c", subcore_axis_name="s",
        num_cores=_NC, num_subcores=_NS)

    @functools.partial(
        pl.kernel,
        mesh=mesh,
        out_type=jax.ShapeDtypeStruct((B, D), jnp.float32),
        scratch_types=[
            pltpu.VMEM((NCHUNK, ROWS), jnp.int32),
            pltpu.VMEM((ROWS, D), jnp.float32),
            pltpu.VMEM((BPW, D), jnp.float32),
            pltpu.SemaphoreType.DMA,
        ],
    )
    def cbow_kernel(idx_hbm, table_hbm, out_hbm, idx_v, rows_v, acc_v, sem):
        wid = lax.axis_index("s") * _NC + lax.axis_index("c")
        pltpu.sync_copy(idx_hbm.at[wid], idx_v)

        @pl.loop(0, NCHUNK)
        def chunk(j):
            pltpu.async_copy(table_hbm.at[idx_v.at[j]], rows_v, sem).wait()
            for be in range(CB):
                for c in range(D // _LANES):
                    acc = rows_v[be * L, pl.ds(c * _LANES, _LANES)]
                    for r in range(1, L):
                        acc = acc + rows_v[be * L + r, pl.ds(c * _LANES, _LANES)]
                    acc_v[j * CB + be, pl.ds(c * _LANES, _LANES)] = acc

        pltpu.sync_copy(acc_v, out_hbm.at[pl.ds(wid * BPW, BPW)])

    return cbow_kernel(idx3, emb_table)


def _ffn_tensorcore(cbow, W1, b1, W2, b2):
    B, D = cbow.shape
    H = W1.shape[1]
    C = W2.shape[1]
    BB = 2048

    def body(x_ref, w1_ref, b1_ref, w2_ref, b2_ref, o_ref):
        h = jnp.dot(x_ref[:], w1_ref[:], preferred_element_type=jnp.float32)
        h = jnp.maximum(h + b1_ref[:], 0.0)
        o_ref[:] = jnp.dot(h, w2_ref[:],
                           preferred_element_type=jnp.float32) + b2_ref[:]

    return pl.pallas_call(
        body,
        grid=(B // BB,),
        in_specs=[
            pl.BlockSpec((BB, D), lambda i: (i, 0)),
            pl.BlockSpec((D, H), lambda i: (0, 0)),
            pl.BlockSpec((1, H), lambda i: (0, 0)),
            pl.BlockSpec((H, C), lambda i: (0, 0)),
            pl.BlockSpec((1, C), lambda i: (0, 0)),
        ],
        out_specs=pl.BlockSpec((BB, C), lambda i: (i, 0)),
        out_shape=jax.ShapeDtypeStruct((B, C), jnp.float32),
    )(cbow, W1, b1.reshape(1, H), W2, b2.reshape(1, C))


def kernel(indices, emb_table, W1, b1, W2, b2):
    cbow = _cbow_sparsecore(indices, emb_table)
    return _ffn_tensorcore(cbow, W1, b1, W2, b2)


# speedup vs baseline: 2.0569x; 2.0569x over previous
import functools

import jax
import jax.numpy as jnp
from jax import lax
from jax.experimental import pallas as pl
from jax.experimental.pallas import tpu as pltpu
from jax.experimental.pallas import tpu_sc as plsc

_NC = 2
_NS = 16
_LANES = 16


def _cbow_sparsecore(indices, emb_table):
    B, L = indices.shape
    V, D = emb_table.shape
    NW = _NC * _NS
    BPW = B // NW
    CB = 2
    ROWS = CB * L
    NCHUNK = BPW // CB

    idx3 = indices.reshape(NW, NCHUNK, ROWS)
    mesh = plsc.VectorSubcoreMesh(
        core_axis_name="c", subcore_axis_name="s",
        num_cores=_NC, num_subcores=_NS)

    @functools.partial(
        pl.kernel,
        mesh=mesh,
        out_type=jax.ShapeDtypeStruct((B, D), jnp.float32),
        scratch_types=[
            pltpu.VMEM((NCHUNK, ROWS), jnp.int32),
            pltpu.VMEM((ROWS, D), jnp.float32),
            pltpu.VMEM((BPW, D), jnp.float32),
            pltpu.SemaphoreType.DMA,
        ],
        compiler_params=pltpu.CompilerParams(use_tc_tiling_on_sc=False),
    )
    def cbow_kernel(idx_hbm, table_hbm, out_hbm, idx_v, rows_v, acc_v, sem):
        wid = lax.axis_index("s") * _NC + lax.axis_index("c")
        pltpu.sync_copy(idx_hbm.at[wid], idx_v)

        @pl.loop(0, NCHUNK)
        def chunk(j):
            pltpu.async_copy(table_hbm.at[idx_v.at[j]], rows_v, sem).wait()
            for be in range(CB):
                for c in range(D // _LANES):
                    acc = rows_v[be * L, pl.ds(c * _LANES, _LANES)]
                    for r in range(1, L):
                        acc = acc + rows_v[be * L + r, pl.ds(c * _LANES, _LANES)]
                    acc_v[j * CB + be, pl.ds(c * _LANES, _LANES)] = acc

        pltpu.sync_copy(acc_v, out_hbm.at[pl.ds(wid * BPW, BPW)])

    return cbow_kernel(idx3, emb_table)


def _ffn_tensorcore(cbow, W1, b1, W2, b2):
    B, D = cbow.shape
    H = W1.shape[1]
    C = W2.shape[1]
    BB = 2048

    def body(x_ref, w1_ref, b1_ref, w2_ref, b2_ref, o_ref):
        h = jnp.dot(x_ref[:], w1_ref[:], preferred_element_type=jnp.float32)
        h = jnp.maximum(h + b1_ref[:], 0.0)
        o_ref[:] = jnp.dot(h, w2_ref[:],
                           preferred_element_type=jnp.float32) + b2_ref[:]

    return pl.pallas_call(
        body,
        grid=(B // BB,),
        in_specs=[
            pl.BlockSpec((BB, D), lambda i: (i, 0)),
            pl.BlockSpec((D, H), lambda i: (0, 0)),
            pl.BlockSpec((1, H), lambda i: (0, 0)),
            pl.BlockSpec((H, C), lambda i: (0, 0)),
            pl.BlockSpec((1, C), lambda i: (0, 0)),
        ],
        out_specs=pl.BlockSpec((BB, C), lambda i: (i, 0)),
        out_shape=jax.ShapeDtypeStruct((B, C), jnp.float32),
    )(cbow, W1, b1.reshape(1, H), W2, b2.reshape(1, C))


def kernel(indices, emb_table, W1, b1, W2, b2):
    cbow = _cbow_sparsecore(indices, emb_table)
    return _ffn_tensorcore(cbow, W1, b1, W2, b2)


# speedup vs baseline: 2.0745x; 1.0085x over previous
import functools

import jax
import jax.numpy as jnp
from jax import lax
from jax.experimental import pallas as pl
from jax.experimental.pallas import tpu as pltpu
from jax.experimental.pallas import tpu_sc as plsc

_NC = 2
_NS = 16
_LANES = 16


def _cbow_sparsecore(indices, emb_table):
    B, L = indices.shape
    V, D = emb_table.shape
    NW = _NC * _NS
    BPW = B // NW
    CB = 2
    ROWS = CB * L
    NCHUNK = BPW // CB

    idx3 = indices.reshape(NW, NCHUNK, ROWS)
    mesh = plsc.VectorSubcoreMesh(
        core_axis_name="c", subcore_axis_name="s",
        num_cores=_NC, num_subcores=_NS)

    NBUF = 4

    @functools.partial(
        pl.kernel,
        mesh=mesh,
        out_type=jax.ShapeDtypeStruct((B, D), jnp.float32),
        scratch_types=[
            pltpu.VMEM((NCHUNK, ROWS), jnp.int32),
            pltpu.VMEM((NBUF, ROWS, D), jnp.float32),
            pltpu.VMEM((BPW, D), jnp.float32),
            [pltpu.SemaphoreType.DMA] * NBUF,
        ],
        compiler_params=pltpu.CompilerParams(use_tc_tiling_on_sc=False),
    )
    def cbow_kernel(idx_hbm, table_hbm, out_hbm, idx_v, rows_v, acc_v, sems):
        wid = lax.axis_index("s") * _NC + lax.axis_index("c")
        pltpu.sync_copy(idx_hbm.at[wid], idx_v)

        def start(j, b):
            pltpu.async_copy(table_hbm.at[idx_v.at[j]], rows_v.at[b], sems[b])

        def finish(j, b):
            pltpu.make_async_copy(
                table_hbm.at[idx_v.at[j]], rows_v.at[b], sems[b]).wait()
            for be in range(CB):
                for c in range(D // _LANES):
                    acc = rows_v[b, be * L, pl.ds(c * _LANES, _LANES)]
                    for r in range(1, L):
                        acc = acc + rows_v[b, be * L + r,
                                           pl.ds(c * _LANES, _LANES)]
                    acc_v[j * CB + be, pl.ds(c * _LANES, _LANES)] = acc

        for b in range(NBUF):
            start(b, b)

        @pl.loop(0, NCHUNK - NBUF, step=NBUF)
        def chunk(j0):
            for b in range(NBUF):
                finish(j0 + b, b)
                start(j0 + b + NBUF, b)

        for b in range(NBUF):
            finish(NCHUNK - NBUF + b, b)

        pltpu.sync_copy(acc_v, out_hbm.at[pl.ds(wid * BPW, BPW)])

    return cbow_kernel(idx3, emb_table)


def _ffn_tensorcore(cbow, W1, b1, W2, b2):
    B, D = cbow.shape
    H = W1.shape[1]
    C = W2.shape[1]
    BB = 2048

    def body(x_ref, w1_ref, b1_ref, w2_ref, b2_ref, o_ref):
        h = jnp.dot(x_ref[:], w1_ref[:], preferred_element_type=jnp.float32)
        h = jnp.maximum(h + b1_ref[:], 0.0)
        o_ref[:] = jnp.dot(h, w2_ref[:],
                           preferred_element_type=jnp.float32) + b2_ref[:]

    return pl.pallas_call(
        body,
        grid=(B // BB,),
        in_specs=[
            pl.BlockSpec((BB, D), lambda i: (i, 0)),
            pl.BlockSpec((D, H), lambda i: (0, 0)),
            pl.BlockSpec((1, H), lambda i: (0, 0)),
            pl.BlockSpec((H, C), lambda i: (0, 0)),
            pl.BlockSpec((1, C), lambda i: (0, 0)),
        ],
        out_specs=pl.BlockSpec((BB, C), lambda i: (i, 0)),
        out_shape=jax.ShapeDtypeStruct((B, C), jnp.float32),
    )(cbow, W1, b1.reshape(1, H), W2, b2.reshape(1, C))


def kernel(indices, emb_table, W1, b1, W2, b2):
    cbow = _cbow_sparsecore(indices, emb_table)
    return _ffn_tensorcore(cbow, W1, b1, W2, b2)


# speedup vs baseline: 2.6111x; 1.2587x over previous
import functools

import jax
import jax.numpy as jnp
from jax import lax
from jax.experimental import pallas as pl
from jax.experimental.pallas import tpu as pltpu
from jax.experimental.pallas import tpu_sc as plsc

_NC = 2
_NS = 16
_LANES = 16


def _cbow_sparsecore(indices, emb_table):
    B, L = indices.shape
    V, D = emb_table.shape
    NW = _NC * _NS
    BPW = B // NW
    NCD = D // _LANES

    mesh = plsc.VectorSubcoreMesh(
        core_axis_name="c", subcore_axis_name="s",
        num_cores=_NC, num_subcores=_NS)

    NBUF = 4

    @functools.partial(
        pl.kernel,
        mesh=mesh,
        out_type=jax.ShapeDtypeStruct((B, D), jnp.float32),
        scratch_types=[
            pltpu.VMEM((BPW, L), jnp.int32),
            pltpu.VMEM((NBUF, L, D), jnp.float32),
            pltpu.VMEM((BPW, D), jnp.float32),
            [pltpu.SemaphoreType.DMA] * NBUF,
        ],
        compiler_params=pltpu.CompilerParams(use_tc_tiling_on_sc=False),
    )
    def cbow_kernel(idx_hbm, table_hbm, out_hbm, idx_v, rows_v, acc_v, sems):
        wid = lax.axis_index("s") * _NC + lax.axis_index("c")
        pltpu.sync_copy(idx_hbm.at[pl.ds(wid * BPW, BPW)], idx_v)

        def start(j, b):
            pltpu.async_copy(table_hbm.at[idx_v.at[j]], rows_v.at[b], sems[b])

        def finish(j, b):
            pltpu.make_async_copy(
                table_hbm.at[idx_v.at[j]], rows_v.at[b], sems[b]).wait()
            accs = [rows_v[b, 0, pl.ds(c * _LANES, _LANES)]
                    for c in range(NCD)]
            for r in range(1, L):
                for c in range(NCD):
                    accs[c] = accs[c] + rows_v[b, r, pl.ds(c * _LANES, _LANES)]
            for c in range(NCD):
                acc_v[j, pl.ds(c * _LANES, _LANES)] = accs[c]

        for b in range(NBUF):
            start(b, b)

        @pl.loop(0, BPW - NBUF, step=NBUF)
        def chunk(j0):
            for b in range(NBUF):
                finish(j0 + b, b)
                start(j0 + b + NBUF, b)

        for b in range(NBUF):
            finish(BPW - NBUF + b, b)

        pltpu.sync_copy(acc_v, out_hbm.at[pl.ds(wid * BPW, BPW)])

    return cbow_kernel(indices, emb_table)


def _ffn_tensorcore(cbow, W1, b1, W2, b2):
    B, D = cbow.shape
    H = W1.shape[1]
    C = W2.shape[1]
    BB = 2048

    def body(x_ref, w1_ref, b1_ref, w2_ref, b2_ref, o_ref):
        h = jnp.dot(x_ref[:], w1_ref[:], preferred_element_type=jnp.float32)
        h = jnp.maximum(h + b1_ref[:], 0.0)
        o_ref[:] = jnp.dot(h, w2_ref[:],
                           preferred_element_type=jnp.float32) + b2_ref[:]

    return pl.pallas_call(
        body,
        grid=(B // BB,),
        in_specs=[
            pl.BlockSpec((BB, D), lambda i: (i, 0)),
            pl.BlockSpec((D, H), lambda i: (0, 0)),
            pl.BlockSpec((1, H), lambda i: (0, 0)),
            pl.BlockSpec((H, C), lambda i: (0, 0)),
            pl.BlockSpec((1, C), lambda i: (0, 0)),
        ],
        out_specs=pl.BlockSpec((BB, C), lambda i: (i, 0)),
        out_shape=jax.ShapeDtypeStruct((B, C), jnp.float32),
    )(cbow, W1, b1.reshape(1, H), W2, b2.reshape(1, C))


def kernel(indices, emb_table, W1, b1, W2, b2):
    cbow = _cbow_sparsecore(indices, emb_table)
    return _ffn_tensorcore(cbow, W1, b1, W2, b2)
